# baseline (speedup 1.0000x reference)
import functools

import jax
import jax.numpy as jnp
from jax import lax
from jax.experimental import pallas as pl
from jax.experimental.pallas import tpu as pltpu
from jax.experimental.pallas import tpu_sc as plsc

_B, _S, _D = 4, 8192, 1024
_C = 32


def _make_sc_kernel():
    info = plsc.get_sparse_core_info()
    nc, ns = info.num_cores, info.num_subcores
    nw = nc * ns
    s_per_w = _S // nw
    n_chunks = s_per_w // _C
    cd = _C * _D

    mesh = plsc.VectorSubcoreMesh(core_axis_name="c", subcore_axis_name="s")

    @functools.partial(
        pl.kernel,
        mesh=mesh,
        out_type=jax.ShapeDtypeStruct((_B * _S * _D,), jnp.float32),
        scratch_types=[
            pltpu.VMEM((cd,), jnp.float32),
            pltpu.VMEM((2, cd), jnp.float32),
            pltpu.SemaphoreType.DMA((2,)),
            pltpu.SemaphoreType.DMA((2,)),
        ],
    )
    def sc_add(x_hbm, pos_hbm, out_hbm, pos_v, x_v, isem, osem):
        wid = lax.axis_index("s") * nc + lax.axis_index("c")
        w_elem = wid * s_per_w * _D

        items = [(c, b) for c in range(n_chunks) for b in range(_B)]
        n_items = len(items)

        def x_off(c, b):
            return w_elem + (b * _S * _D) + c * cd

        def start_in(i):
            c, b = items[i]
            pltpu.async_copy(x_hbm.at[pl.ds(x_off(c, b), cd)],
                             x_v.at[i % 2], isem.at[i % 2])

        def wait_in(i):
            c, b = items[i]
            pltpu.make_async_copy(x_hbm.at[pl.ds(x_off(c, b), cd)],
                                  x_v.at[i % 2], isem.at[i % 2]).wait()

        def start_out(i):
            c, b = items[i]
            pltpu.async_copy(x_v.at[i % 2],
                             out_hbm.at[pl.ds(x_off(c, b), cd)],
                             osem.at[i % 2])

        def wait_out(i):
            c, b = items[i]
            pltpu.make_async_copy(x_v.at[i % 2],
                                  out_hbm.at[pl.ds(x_off(c, b), cd)],
                                  osem.at[i % 2]).wait()

        start_in(0)
        for i, (c, b) in enumerate(items):
            if i + 1 < n_items:
                if i >= 1:
                    wait_out(i - 1)
                start_in(i + 1)
            if b == 0:
                pltpu.sync_copy(pos_hbm.at[pl.ds(w_elem + c * cd, cd)], pos_v)
            wait_in(i)

            xbuf = x_v.at[i % 2]

            @plsc.parallel_loop(0, cd, 16, unroll=8)
            def _(j):
                plsc.addupdate(xbuf.at[pl.ds(j, 16)], pos_v[pl.ds(j, 16)])

            start_out(i)
        wait_out(n_items - 2)
        wait_out(n_items - 1)

    return sc_add


_sc_add = _make_sc_kernel()


def kernel(x, pos_table):
    b, s, d = x.shape
    out = _sc_add(x.reshape(-1), pos_table.reshape(-1))
    return out.reshape(b, s, d)


# speedup vs baseline: 1.2141x; 1.2141x over previous
import functools

import jax
import jax.numpy as jnp
from jax import lax
from jax.experimental import pallas as pl
from jax.experimental.pallas import tpu as pltpu
from jax.experimental.pallas import tpu_sc as plsc

_B, _S, _D = 4, 8192, 1024
_C = 16
_N_SC = 6144
_RB = 512


def _make_sc_kernel(n_rows, row0):
    info = plsc.get_sparse_core_info()
    nc, ns = info.num_cores, info.num_subcores
    nw = nc * ns
    rpw = n_rows // nw
    n_chunks = rpw // _C
    cd = _C * _D

    mesh = plsc.VectorSubcoreMesh(core_axis_name="c", subcore_axis_name="s")

    @functools.partial(
        pl.kernel,
        mesh=mesh,
        out_type=jax.ShapeDtypeStruct((n_rows * _D,), jnp.float32),
        scratch_types=[
            pltpu.VMEM((2, cd), jnp.float32),
            pltpu.VMEM((2, cd), jnp.float32),
            pltpu.SemaphoreType.DMA((2,)),
            pltpu.SemaphoreType.DMA((2,)),
            pltpu.SemaphoreType.DMA((2,)),
        ],
    )
    def sc_add(x_hbm, pos_hbm, out_hbm, pos_v, x_v, psem, isem, osem):
        wid = lax.axis_index("s") * nc + lax.axis_index("c")
        base = wid * rpw

        def x_off(c):
            return (row0 + base + c * _C) * _D

        def o_off(c):
            return (base + c * _C) * _D

        def pos_off(c):
            return lax.rem(row0 + base + c * _C, _S) * _D

        def start_pos(c):
            pltpu.async_copy(pos_hbm.at[pl.ds(pos_off(c), cd)],
                             pos_v.at[c % 2], psem.at[c % 2])

        def wait_pos(c):
            pltpu.make_async_copy(pos_hbm.at[pl.ds(pos_off(c), cd)],
                                  pos_v.at[c % 2], psem.at[c % 2]).wait()

        def start_in(c):
            pltpu.async_copy(x_hbm.at[pl.ds(x_off(c), cd)],
                             x_v.at[c % 2], isem.at[c % 2])

        def wait_in(c):
            pltpu.make_async_copy(x_hbm.at[pl.ds(x_off(c), cd)],
                                  x_v.at[c % 2], isem.at[c % 2]).wait()

        def start_out(c):
            pltpu.async_copy(x_v.at[c % 2],
                             out_hbm.at[pl.ds(o_off(c), cd)],
                             osem.at[c % 2])

        def wait_out(c):
            pltpu.make_async_copy(x_v.at[c % 2],
                                  out_hbm.at[pl.ds(o_off(c), cd)],
                                  osem.at[c % 2]).wait()

        start_pos(0)
        start_in(0)
        for c in range(n_chunks):
            if c + 1 < n_chunks:
                if c >= 1:
                    wait_out(c - 1)
                start_in(c + 1)
                start_pos(c + 1)
            wait_pos(c)
            wait_in(c)

            xbuf = x_v.at[c % 2]
            pbuf = pos_v.at[c % 2]

            @plsc.parallel_loop(0, cd, 16, unroll=8)
            def _(j):
                plsc.addupdate(xbuf.at[pl.ds(j, 16)], pbuf[pl.ds(j, 16)])

            start_out(c)
        if n_chunks >= 2:
            wait_out(n_chunks - 2)
        wait_out(n_chunks - 1)

    return sc_add


_sc_add = _make_sc_kernel(_N_SC, _B * _S - _N_SC)


def _tc_body(x_ref, p_ref, o_ref):
    o_ref[...] = x_ref[...] + p_ref[...]


def kernel(x, pos_table):
    b, s, d = x.shape
    rows = b * s
    n_tc = rows - _N_SC
    xf = x.reshape(rows, d)
    ntb = n_tc // _RB
    npb = s // _RB
    tc_out = pl.pallas_call(
        _tc_body,
        grid=(ntb,),
        in_specs=[
            pl.BlockSpec((_RB, d), lambda i: (i, 0)),
            pl.BlockSpec((_RB, d), lambda i, npb=npb: (i % npb, 0)),
        ],
        out_specs=pl.BlockSpec((_RB, d), lambda i: (i, 0)),
        out_shape=jax.ShapeDtypeStruct((n_tc, d), x.dtype),
    )(xf, pos_table)
    sc_out = _sc_add(x.reshape(-1), pos_table.reshape(-1)).reshape(_N_SC, d)
    return jnp.concatenate([tc_out, sc_out], axis=0).reshape(b, s, d)


# speedup vs baseline: 2.8417x; 2.3407x over previous
import functools

import jax
import jax.numpy as jnp
from jax import lax
from jax.experimental import pallas as pl
from jax.experimental.pallas import tpu as pltpu
from jax.experimental.pallas import tpu_sc as plsc

_B, _S, _D = 4, 8192, 1024
_ROWS = _B * _S
_C = 16
_N_SC = 6144
_N_TC = _ROWS - _N_SC
_RB = 512


def _make_sc_kernel():
    info = plsc.get_sparse_core_info()
    nc, ns = info.num_cores, info.num_subcores
    nw = nc * ns
    rpw = _N_SC // nw
    n_chunks = rpw // _C

    mesh = plsc.VectorSubcoreMesh(core_axis_name="c", subcore_axis_name="s")

    @functools.partial(
        pl.kernel,
        mesh=mesh,
        out_type=jax.ShapeDtypeStruct((_N_SC, _D), jnp.float32),
        compiler_params=pltpu.CompilerParams(use_tc_tiling_on_sc=True),
        scratch_types=[
            pltpu.VMEM((2, _C, _D), jnp.float32),
            pltpu.VMEM((2, _C, _D), jnp.float32),
            pltpu.SemaphoreType.DMA((2,)),
            pltpu.SemaphoreType.DMA((2,)),
            pltpu.SemaphoreType.DMA((2,)),
        ],
    )
    def sc_add(x_hbm, pos_hbm, out_hbm, pos_v, x_v, psem, isem, osem):
        wid = lax.axis_index("s") * nc + lax.axis_index("c")
        base = wid * rpw

        def row0(c):
            return base + c * _C

        def s0(c):
            return lax.rem(_N_TC + row0(c), _S)

        def start_pos(c):
            pltpu.async_copy(pos_hbm.at[pl.ds(s0(c), _C), :],
                             pos_v.at[c % 2], psem.at[c % 2])

        def wait_pos(c):
            pltpu.make_async_copy(pos_hbm.at[pl.ds(s0(c), _C), :],
                                  pos_v.at[c % 2], psem.at[c % 2]).wait()

        def start_in(c):
            pltpu.async_copy(x_hbm.at[pl.ds(_N_TC + row0(c), _C), :],
                             x_v.at[c % 2], isem.at[c % 2])

        def wait_in(c):
            pltpu.make_async_copy(x_hbm.at[pl.ds(_N_TC + row0(c), _C), :],
                                  x_v.at[c % 2], isem.at[c % 2]).wait()

        def start_out(c):
            pltpu.async_copy(x_v.at[c % 2],
                             out_hbm.at[pl.ds(row0(c), _C), :],
                             osem.at[c % 2])

        def wait_out(c):
            pltpu.make_async_copy(x_v.at[c % 2],
                                  out_hbm.at[pl.ds(row0(c), _C), :],
                                  osem.at[c % 2]).wait()

        start_pos(0)
        start_in(0)
        for c in range(n_chunks):
            if c + 1 < n_chunks:
                if c >= 1:
                    wait_out(c - 1)
                start_in(c + 1)
                start_pos(c + 1)
            wait_pos(c)
            wait_in(c)

            xbuf = x_v.at[c % 2]
            pbuf = pos_v.at[c % 2]
            for r in range(_C):
                @plsc.parallel_loop(0, _D, 16, unroll=8)
                def _(j):
                    plsc.addupdate(xbuf.at[r, pl.ds(j, 16)],
                                   pbuf[r, pl.ds(j, 16)])

            start_out(c)
        if n_chunks >= 2:
            wait_out(n_chunks - 2)
        wait_out(n_chunks - 1)

    return sc_add


_sc_add = _make_sc_kernel()


def _tc_body(x_ref, p_ref, o_ref):
    o_ref[...] = x_ref[...] + p_ref[...]


def kernel(x, pos_table):
    b, s, d = x.shape
    xf = x.reshape(_ROWS, d)
    npb = s // _RB
    tc_out = pl.pallas_call(
        _tc_body,
        grid=(_N_TC // _RB,),
        in_specs=[
            pl.BlockSpec((_RB, d), lambda i: (i, 0)),
            pl.BlockSpec((_RB, d), lambda i, npb=npb: (i % npb, 0)),
        ],
        out_specs=pl.BlockSpec((_RB, d), lambda i: (i, 0)),
        out_shape=jax.ShapeDtypeStruct((_ROWS, d), x.dtype),
    )(xf, pos_table)
    sc_out = _sc_add(xf, pos_table)
    out = lax.dynamic_update_slice(tc_out, sc_out, (_N_TC, 0))
    return out.reshape(b, s, d)


# speedup vs baseline: 2.9686x; 1.0447x over previous
import functools

import jax
import jax.numpy as jnp
from jax import lax
from jax.experimental import pallas as pl
from jax.experimental.pallas import tpu as pltpu
from jax.experimental.pallas import tpu_sc as plsc

_B, _S, _D = 4, 8192, 1024
_ROWS = _B * _S
_C = 16
_N_SC = 3072
_N_TC = _ROWS - _N_SC
_RB = 512


def _make_sc_kernel():
    info = plsc.get_sparse_core_info()
    nc, ns = info.num_cores, info.num_subcores
    nw = nc * ns
    rpw = _N_SC // nw
    n_chunks = rpw // _C

    mesh = plsc.VectorSubcoreMesh(core_axis_name="c", subcore_axis_name="s")

    @functools.partial(
        pl.kernel,
        mesh=mesh,
        out_type=jax.ShapeDtypeStruct((_N_SC, _D), jnp.float32),
        compiler_params=pltpu.CompilerParams(use_tc_tiling_on_sc=True),
        scratch_types=[
            pltpu.VMEM((2, _C, _D), jnp.float32),
            pltpu.VMEM((2, _C, _D), jnp.float32),
            pltpu.SemaphoreType.DMA((2,)),
            pltpu.SemaphoreType.DMA((2,)),
            pltpu.SemaphoreType.DMA((2,)),
        ],
    )
    def sc_add(x_hbm, pos_hbm, out_hbm, pos_v, x_v, psem, isem, osem):
        wid = lax.axis_index("s") * nc + lax.axis_index("c")
        base = wid * rpw

        def row0(c):
            return base + c * _C

        def s0(c):
            return lax.rem(_N_TC + row0(c), _S)

        def start_pos(c):
            pltpu.async_copy(pos_hbm.at[pl.ds(s0(c), _C), :],
                             pos_v.at[c % 2], psem.at[c % 2])

        def wait_pos(c):
            pltpu.make_async_copy(pos_hbm.at[pl.ds(s0(c), _C), :],
                                  pos_v.at[c % 2], psem.at[c % 2]).wait()

        def start_in(c):
            pltpu.async_copy(x_hbm.at[pl.ds(_N_TC + row0(c), _C), :],
                             x_v.at[c % 2], isem.at[c % 2])

        def wait_in(c):
            pltpu.make_async_copy(x_hbm.at[pl.ds(_N_TC + row0(c), _C), :],
                                  x_v.at[c % 2], isem.at[c % 2]).wait()

        def start_out(c):
            pltpu.async_copy(x_v.at[c % 2],
                             out_hbm.at[pl.ds(row0(c), _C), :],
                             osem.at[c % 2])

        def wait_out(c):
            pltpu.make_async_copy(x_v.at[c % 2],
                                  out_hbm.at[pl.ds(row0(c), _C), :],
                                  osem.at[c % 2]).wait()

        start_pos(0)
        start_in(0)
        for c in range(n_chunks):
            if c + 1 < n_chunks:
                if c >= 1:
                    wait_out(c - 1)
                start_in(c + 1)
                start_pos(c + 1)
            wait_pos(c)
            wait_in(c)

            xbuf = x_v.at[c % 2]
            pbuf = pos_v.at[c % 2]
            for r in range(_C):
                @plsc.parallel_loop(0, _D, 16, unroll=8)
                def _(j):
                    plsc.addupdate(xbuf.at[r, pl.ds(j, 16)],
                                   pbuf[r, pl.ds(j, 16)])

            start_out(c)
        if n_chunks >= 2:
            wait_out(n_chunks - 2)
        wait_out(n_chunks - 1)

    return sc_add


_sc_add = _make_sc_kernel()


def _tc_body(x_ref, p_ref, o_ref):
    o_ref[...] = x_ref[...] + p_ref[...]


def kernel(x, pos_table):
    b, s, d = x.shape
    xf = x.reshape(_ROWS, d)
    npb = s // _RB
    tc_out = pl.pallas_call(
        _tc_body,
        grid=(_N_TC // _RB,),
        in_specs=[
            pl.BlockSpec((_RB, d), lambda i: (i, 0)),
            pl.BlockSpec((_RB, d), lambda i, npb=npb: (i % npb, 0)),
        ],
        out_specs=pl.BlockSpec((_RB, d), lambda i: (i, 0)),
        out_shape=jax.ShapeDtypeStruct((_ROWS, d), x.dtype),
    )(xf, pos_table)
    sc_out = _sc_add(xf, pos_table)
    out = lax.dynamic_update_slice(tc_out, sc_out, (_N_TC, 0))
    return out.reshape(b, s, d)
